# Initial kernel scaffold; baseline (speedup 1.0000x reference)
import jax
import jax.numpy as jnp
from jax.experimental import pallas as pl


def kernel(x, edge_index, W1, b1, W2, b2):
    raise NotImplementedError("write your pallas kernel here")


# speedup vs baseline: 2.8570x; 2.8570x over previous
import functools

import jax
import jax.numpy as jnp
from jax import lax
from jax.experimental import pallas as pl
from jax.experimental.pallas import tpu as pltpu
from jax.experimental.pallas import tpu_sc as plsc

N_NODES = 10000
N_EDGES = 320000
D = 128

NC = 2
NS = 16
L = 16

R = 10112
CHUNK = 128
CH_PER_W = 80
CPB = 16
NB = CH_PER_W // CPB
E_PAD = NC * NS * CH_PER_W * CHUNK
ROWS_PER_TILE = R // NS


def _seg_sum_body(x_hbm, srcs_hbm, dsts_hbm, out_hbm,
                  src_v, dst_v, rows0, rows1, acc, sem0, sem1):
    c = lax.axis_index("c")
    s = lax.axis_index("s")
    wid = c * NS + s

    zero = jnp.zeros((L,), jnp.float32)

    def zrow(i, _):
        for j in range(D // L):
            rows0[i, pl.ds(j * L, L)] = zero
        return 0

    lax.fori_loop(0, CHUNK, zrow, 0)
    row0 = s * ROWS_PER_TILE
    for k in range(ROWS_PER_TILE // CHUNK):
        pltpu.sync_copy(rows0, acc.at[pl.ds(row0 + k * CHUNK, CHUNK)])
    rem = ROWS_PER_TILE % CHUNK
    if rem:
        pltpu.sync_copy(rows0.at[pl.ds(0, rem)],
                        acc.at[pl.ds(row0 + ROWS_PER_TILE - rem, rem)])
    plsc.subcore_barrier()

    def chunk_body(j, _):
        pltpu.async_copy(x_hbm.at[src_v.at[j]], rows0, sem0).wait()
        pltpu.sync_copy(rows0, acc.at[dst_v.at[j]], add=True)
        return 0

    def block_body(b, _):
        base = wid * CH_PER_W + b * CPB
        pltpu.sync_copy(srcs_hbm.at[pl.ds(base, CPB)], src_v)
        pltpu.sync_copy(dsts_hbm.at[pl.ds(base, CPB)], dst_v)
        lax.fori_loop(0, CPB, chunk_body, 0)
        return 0

    lax.fori_loop(0, NB, block_body, 0)
    plsc.subcore_barrier()

    for k in range(ROWS_PER_TILE // CHUNK):
        r = row0 + k * CHUNK
        pltpu.sync_copy(acc.at[pl.ds(r, CHUNK)], rows0)
        pltpu.sync_copy(rows0, out_hbm.at[c, pl.ds(r, CHUNK)])
    if rem:
        r = row0 + ROWS_PER_TILE - rem
        pltpu.sync_copy(acc.at[pl.ds(r, rem)], rows1.at[pl.ds(0, rem)])
        pltpu.sync_copy(rows1.at[pl.ds(0, rem)], out_hbm.at[c, pl.ds(r, rem)])


_seg_sum = pl.kernel(
    _seg_sum_body,
    out_type=jax.ShapeDtypeStruct((NC, R, D), jnp.float32),
    mesh=plsc.VectorSubcoreMesh(core_axis_name="c", subcore_axis_name="s",
                                num_cores=NC, num_subcores=NS),
    scratch_types=[
        pltpu.VMEM((CPB, CHUNK), jnp.int32),
        pltpu.VMEM((CPB, CHUNK), jnp.int32),
        pltpu.VMEM((CHUNK, D), jnp.float32),
        pltpu.VMEM((CHUNK, D), jnp.float32),
        pltpu.VMEM_SHARED((R, D), jnp.float32),
        pltpu.SemaphoreType.DMA,
        pltpu.SemaphoreType.DMA,
    ],
)


def _mlp_body(x_ref, p_ref, w_ref, b_ref, o_ref, *, relu):
    acc = x_ref[...] + p_ref[0] + p_ref[1]
    h = jnp.dot(acc, w_ref[...], preferred_element_type=jnp.float32)
    h = h + b_ref[...]
    if relu:
        h = jnp.maximum(h, 0.0)
    o_ref[...] = h


def _mlp(x, partials, w, b2d, relu):
    blk = 1000
    grid = (N_NODES // blk,)
    return pl.pallas_call(
        functools.partial(_mlp_body, relu=relu),
        grid=grid,
        in_specs=[
            pl.BlockSpec((blk, D), lambda i: (i, 0)),
            pl.BlockSpec((NC, blk, D), lambda i: (0, i, 0)),
            pl.BlockSpec((D, D), lambda i: (0, 0)),
            pl.BlockSpec((1, D), lambda i: (0, 0)),
        ],
        out_specs=pl.BlockSpec((blk, D), lambda i: (i, 0)),
        out_shape=jax.ShapeDtypeStruct((N_NODES, D), jnp.float32),
    )(x, partials, w, b2d)


def kernel(x, edge_index, W1, b1, W2, b2):
    src = edge_index[0].astype(jnp.int32)
    dst = edge_index[1].astype(jnp.int32)
    pad = E_PAD - N_EDGES
    srcs = jnp.concatenate([src, jnp.zeros((pad,), jnp.int32)]).reshape(-1, CHUNK)
    dsts = jnp.concatenate([dst, jnp.full((pad,), N_NODES, jnp.int32)]).reshape(-1, CHUNK)
    b1r = b1.reshape(1, D)
    b2r = b2.reshape(1, D)

    p1 = _seg_sum(x, srcs, dsts)
    h = _mlp(x, p1, W1, b1r, relu=True)
    p2 = _seg_sum(h, srcs, dsts)
    return _mlp(h, p2, W2, b2r, relu=False)


# speedup vs baseline: 3.0992x; 1.0848x over previous
import functools

import jax
import jax.numpy as jnp
from jax import lax
from jax.experimental import pallas as pl
from jax.experimental.pallas import tpu as pltpu
from jax.experimental.pallas import tpu_sc as plsc

N_NODES = 10000
N_EDGES = 320000
D = 128

NC = 2
NS = 16
L = 16

R = 10112
CHUNK = 128
CH_PER_W = 80
CPB = 16
NB = CH_PER_W // CPB
E_PAD = NC * NS * CH_PER_W * CHUNK
ROWS_PER_TILE = R // NS


def _seg_sum_body(x_hbm, srcs_hbm, dsts_hbm, out_hbm,
                  src_a, dst_a, src_b, dst_b, rows0, rows1, acc,
                  g0, g1, s0, s1, ia, ib):
    c = lax.axis_index("c")
    s = lax.axis_index("s")
    wid = c * NS + s

    zero = jnp.zeros((L,), jnp.float32)

    def zrow(i, _):
        for j in range(D // L):
            rows0[i, pl.ds(j * L, L)] = zero
        return 0

    lax.fori_loop(0, CHUNK, zrow, 0)
    row0 = s * ROWS_PER_TILE
    for k in range(ROWS_PER_TILE // CHUNK):
        pltpu.sync_copy(rows0, acc.at[pl.ds(row0 + k * CHUNK, CHUNK)])
    rem = ROWS_PER_TILE % CHUNK
    if rem:
        pltpu.sync_copy(rows0.at[pl.ds(0, rem)],
                        acc.at[pl.ds(row0 + ROWS_PER_TILE - rem, rem)])
    plsc.subcore_barrier()

    def stage(b, sv, dv, sem):
        base = wid * CH_PER_W + b * CPB
        pltpu.async_copy(srcs_hbm.at[pl.ds(base, CPB)], sv, sem)
        pltpu.async_copy(dsts_hbm.at[pl.ds(base, CPB)], dv, sem)

    def wait_stage(sv, dv, sem):
        pltpu.make_async_copy(srcs_hbm.at[pl.ds(0, CPB)], sv, sem).wait()
        pltpu.make_async_copy(dsts_hbm.at[pl.ds(0, CPB)], dv, sem).wait()

    stage(0, src_a, dst_a, ia)
    for b in range(NB):
        even = b % 2 == 0
        sv, dv, siv = (src_a, dst_a, ia) if even else (src_b, dst_b, ib)
        wait_stage(sv, dv, siv)
        if b + 1 < NB:
            nsv, ndv, nsem = (src_b, dst_b, ib) if even else (src_a, dst_a, ia)
            stage(b + 1, nsv, ndv, nsem)
        pltpu.async_copy(x_hbm.at[sv.at[0]], rows0, g0)
        pltpu.async_copy(x_hbm.at[sv.at[1]], rows1, g1)

        def pair(j, _, sv=sv, dv=dv):
            pltpu.make_async_copy(x_hbm.at[sv.at[j]], rows0, g0).wait()
            pltpu.async_copy(rows0, acc.at[dv.at[j]], s0, add=True)
            pltpu.make_async_copy(x_hbm.at[sv.at[j + 1]], rows1, g1).wait()
            pltpu.async_copy(rows1, acc.at[dv.at[j + 1]], s1, add=True)
            pltpu.make_async_copy(rows0, acc.at[dv.at[j]], s0).wait()
            jn2 = jnp.minimum(j + 2, CPB - 1)
            nx0 = pltpu.make_async_copy(x_hbm.at[sv.at[jn2]], rows0, g0)

            @pl.when(j + 2 < CPB)
            def _():
                nx0.start()

            pltpu.make_async_copy(rows1, acc.at[dv.at[j + 1]], s1).wait()
            jn3 = jnp.minimum(j + 3, CPB - 1)
            nx1 = pltpu.make_async_copy(x_hbm.at[sv.at[jn3]], rows1, g1)

            @pl.when(j + 3 < CPB)
            def _():
                nx1.start()

            return 0

        lax.fori_loop(0, CPB // 2, lambda i, car: pair(2 * i, car), 0)
    plsc.subcore_barrier()

    for k in range(ROWS_PER_TILE // CHUNK):
        r = row0 + k * CHUNK
        pltpu.sync_copy(acc.at[pl.ds(r, CHUNK)], rows0)
        pltpu.sync_copy(rows0, out_hbm.at[c, pl.ds(r, CHUNK)])
    if rem:
        r = row0 + ROWS_PER_TILE - rem
        pltpu.sync_copy(acc.at[pl.ds(r, rem)], rows1.at[pl.ds(0, rem)])
        pltpu.sync_copy(rows1.at[pl.ds(0, rem)], out_hbm.at[c, pl.ds(r, rem)])


_seg_sum = pl.kernel(
    _seg_sum_body,
    out_type=jax.ShapeDtypeStruct((NC, R, D), jnp.float32),
    mesh=plsc.VectorSubcoreMesh(core_axis_name="c", subcore_axis_name="s",
                                num_cores=NC, num_subcores=NS),
    scratch_types=[
        pltpu.VMEM((CPB, CHUNK), jnp.int32),
        pltpu.VMEM((CPB, CHUNK), jnp.int32),
        pltpu.VMEM((CPB, CHUNK), jnp.int32),
        pltpu.VMEM((CPB, CHUNK), jnp.int32),
        pltpu.VMEM((CHUNK, D), jnp.float32),
        pltpu.VMEM((CHUNK, D), jnp.float32),
        pltpu.VMEM_SHARED((R, D), jnp.float32),
        pltpu.SemaphoreType.DMA,
        pltpu.SemaphoreType.DMA,
        pltpu.SemaphoreType.DMA,
        pltpu.SemaphoreType.DMA,
        pltpu.SemaphoreType.DMA,
        pltpu.SemaphoreType.DMA,
    ],
)


def _mlp_body(x_ref, p_ref, w_ref, b_ref, o_ref, *, relu):
    acc = x_ref[...] + p_ref[0] + p_ref[1]
    h = jnp.dot(acc, w_ref[...], preferred_element_type=jnp.float32)
    h = h + b_ref[...]
    if relu:
        h = jnp.maximum(h, 0.0)
    o_ref[...] = h


def _mlp(x, partials, w, b2d, relu):
    blk = 1000
    grid = (N_NODES // blk,)
    return pl.pallas_call(
        functools.partial(_mlp_body, relu=relu),
        grid=grid,
        in_specs=[
            pl.BlockSpec((blk, D), lambda i: (i, 0)),
            pl.BlockSpec((NC, blk, D), lambda i: (0, i, 0)),
            pl.BlockSpec((D, D), lambda i: (0, 0)),
            pl.BlockSpec((1, D), lambda i: (0, 0)),
        ],
        out_specs=pl.BlockSpec((blk, D), lambda i: (i, 0)),
        out_shape=jax.ShapeDtypeStruct((N_NODES, D), jnp.float32),
    )(x, partials, w, b2d)


def kernel(x, edge_index, W1, b1, W2, b2):
    src = edge_index[0].astype(jnp.int32)
    dst = edge_index[1].astype(jnp.int32)
    pad = E_PAD - N_EDGES
    srcs = jnp.concatenate([src, jnp.zeros((pad,), jnp.int32)]).reshape(-1, CHUNK)
    dsts = jnp.concatenate([dst, jnp.full((pad,), N_NODES, jnp.int32)]).reshape(-1, CHUNK)
    b1r = b1.reshape(1, D)
    b2r = b2.reshape(1, D)

    p1 = _seg_sum(x, srcs, dsts)
    h = _mlp(x, p1, W1, b1r, relu=True)
    p2 = _seg_sum(h, srcs, dsts)
    return _mlp(h, p2, W2, b2r, relu=False)


# speedup vs baseline: 10.1015x; 3.2594x over previous
import functools

import jax
import jax.numpy as jnp
from jax import lax
from jax.experimental import pallas as pl
from jax.experimental.pallas import tpu as pltpu
from jax.experimental.pallas import tpu_sc as plsc

N_NODES = 10000
N_EDGES = 320000
D = 128

NC = 2
NS = 16
L = 16

R = 10112
CHUNK = 128
CH_PER_W = 80
CPB = 16
NB = CH_PER_W // CPB
E_PAD = NC * NS * CH_PER_W * CHUNK
ROWS_PER_TILE = R // NS


def _seg_sum_body(x_hbm, srcs_hbm, dsts_hbm, out_hbm,
                  src_a, dst_a, src_b, dst_b, rows0, rows1, acc,
                  g0, g1, s0, s1, ia, ib):
    c = lax.axis_index("c")
    s = lax.axis_index("s")
    wid = c * NS + s

    zero = jnp.zeros((L,), jnp.float32)

    def zrow(i, _):
        for j in range(D // L):
            rows0[i, pl.ds(j * L, L)] = zero
        return 0

    lax.fori_loop(0, CHUNK, zrow, 0)
    row0 = s * ROWS_PER_TILE
    for k in range(ROWS_PER_TILE // CHUNK):
        pltpu.sync_copy(rows0, acc.at[pl.ds(row0 + k * CHUNK, CHUNK)])
    rem = ROWS_PER_TILE % CHUNK
    if rem:
        pltpu.sync_copy(rows0.at[pl.ds(0, rem)],
                        acc.at[pl.ds(row0 + ROWS_PER_TILE - rem, rem)])
    plsc.subcore_barrier()

    def stage(b, sv, dv, sem):
        base = wid * CH_PER_W + b * CPB
        pltpu.async_copy(srcs_hbm.at[pl.ds(base, CPB)], sv, sem)
        pltpu.async_copy(dsts_hbm.at[pl.ds(base, CPB)], dv, sem)

    def wait_stage(sv, dv, sem):
        pltpu.make_async_copy(srcs_hbm.at[pl.ds(0, CPB)], sv, sem).wait()
        pltpu.make_async_copy(dsts_hbm.at[pl.ds(0, CPB)], dv, sem).wait()

    stage(0, src_a, dst_a, ia)
    for b in range(NB):
        even = b % 2 == 0
        sv, dv, siv = (src_a, dst_a, ia) if even else (src_b, dst_b, ib)
        wait_stage(sv, dv, siv)
        if b + 1 < NB:
            nsv, ndv, nsem = (src_b, dst_b, ib) if even else (src_a, dst_a, ia)
            stage(b + 1, nsv, ndv, nsem)
        pltpu.async_copy(x_hbm.at[sv.at[0]], rows0, g0)
        pltpu.async_copy(x_hbm.at[sv.at[1]], rows1, g1)

        def pair(j, _, sv=sv, dv=dv):
            pltpu.make_async_copy(x_hbm.at[sv.at[j]], rows0, g0).wait()
            pltpu.async_copy(rows0, acc.at[dv.at[j]], s0, add=True)
            pltpu.make_async_copy(x_hbm.at[sv.at[j + 1]], rows1, g1).wait()
            pltpu.async_copy(rows1, acc.at[dv.at[j + 1]], s1, add=True)
            pltpu.make_async_copy(rows0, acc.at[dv.at[j]], s0).wait()
            jn2 = jnp.minimum(j + 2, CPB - 1)
            nx0 = pltpu.make_async_copy(x_hbm.at[sv.at[jn2]], rows0, g0)

            @pl.when(j + 2 < CPB)
            def _():
                nx0.start()

            pltpu.make_async_copy(rows1, acc.at[dv.at[j + 1]], s1).wait()
            jn3 = jnp.minimum(j + 3, CPB - 1)
            nx1 = pltpu.make_async_copy(x_hbm.at[sv.at[jn3]], rows1, g1)

            @pl.when(j + 3 < CPB)
            def _():
                nx1.start()

            return 0

        lax.fori_loop(0, CPB // 2, lambda i, car: pair(2 * i, car), 0)
    plsc.subcore_barrier()

    for k in range(ROWS_PER_TILE // CHUNK):
        r = row0 + k * CHUNK
        pltpu.sync_copy(acc.at[pl.ds(r, CHUNK)], rows0)
        pltpu.sync_copy(rows0, out_hbm.at[c, pl.ds(r, CHUNK)])
    if rem:
        r = row0 + ROWS_PER_TILE - rem
        pltpu.sync_copy(acc.at[pl.ds(r, rem)], rows1.at[pl.ds(0, rem)])
        pltpu.sync_copy(rows1.at[pl.ds(0, rem)], out_hbm.at[c, pl.ds(r, rem)])


_seg_sum = pl.kernel(
    _seg_sum_body,
    out_type=jax.ShapeDtypeStruct((NC, R, D), jnp.float32),
    mesh=plsc.VectorSubcoreMesh(core_axis_name="c", subcore_axis_name="s",
                                num_cores=NC, num_subcores=NS),
    scratch_types=[
        pltpu.VMEM((CPB, CHUNK), jnp.int32),
        pltpu.VMEM((CPB, CHUNK), jnp.int32),
        pltpu.VMEM((CPB, CHUNK), jnp.int32),
        pltpu.VMEM((CPB, CHUNK), jnp.int32),
        pltpu.VMEM((CHUNK, D), jnp.float32),
        pltpu.VMEM((CHUNK, D), jnp.float32),
        pltpu.VMEM_SHARED((R, D), jnp.float32),
        pltpu.SemaphoreType.DMA,
        pltpu.SemaphoreType.DMA,
        pltpu.SemaphoreType.DMA,
        pltpu.SemaphoreType.DMA,
        pltpu.SemaphoreType.DMA,
        pltpu.SemaphoreType.DMA,
    ],
)


def _mlp_body(x_ref, p_ref, w_ref, b_ref, o_ref, *, relu):
    acc = x_ref[...] + p_ref[0] + p_ref[1]
    h = jnp.dot(acc, w_ref[...], preferred_element_type=jnp.float32)
    h = h + b_ref[...]
    if relu:
        h = jnp.maximum(h, 0.0)
    o_ref[...] = h


def _mlp(x, partials, w, b2d, relu):
    blk = 1000
    grid = (N_NODES // blk,)
    return pl.pallas_call(
        functools.partial(_mlp_body, relu=relu),
        grid=grid,
        in_specs=[
            pl.BlockSpec((blk, D), lambda i: (i, 0)),
            pl.BlockSpec((NC, blk, D), lambda i: (0, i, 0)),
            pl.BlockSpec((D, D), lambda i: (0, 0)),
            pl.BlockSpec((1, D), lambda i: (0, 0)),
        ],
        out_specs=pl.BlockSpec((blk, D), lambda i: (i, 0)),
        out_shape=jax.ShapeDtypeStruct((N_NODES, D), jnp.float32),
    )(x, partials, w, b2d)


def kernel(x, edge_index, W1, b1, W2, b2):
    src = edge_index[0].astype(jnp.int32)
    dst = edge_index[1].astype(jnp.int32)
    pad = E_PAD - N_EDGES
    pad_ar = jnp.arange(pad, dtype=jnp.int32)
    srcs = jnp.concatenate([src, pad_ar % N_NODES]).reshape(-1, CHUNK)
    dsts = jnp.concatenate([dst, N_NODES + pad_ar % (R - N_NODES)]).reshape(-1, CHUNK)
    b1r = b1.reshape(1, D)
    b2r = b2.reshape(1, D)

    p1 = _seg_sum(x, srcs, dsts)
    h = _mlp(x, p1, W1, b1r, relu=True)
    p2 = _seg_sum(h, srcs, dsts)
    return _mlp(h, p2, W2, b2r, relu=False)
